# baseline (speedup 1.0000x reference)
import jax
import jax.numpy as jnp
from jax.experimental import pallas as pl
from jax.experimental.pallas import tpu as pltpu

_BLK = 256
_DEPTH = 8


def _block_tau(s, xaug, xr, mi):
    adjb = jnp.where(s > 0.0, 1.0, 0.0).astype(jnp.bfloat16)
    z = jnp.dot(adjb, xaug, preferred_element_type=jnp.float32)
    d = xr.shape[1]
    agg = z[:, :d]
    t2 = z[:, d:d + 1]
    deg0 = z[:, d + 1:d + 2]
    sqr = jnp.sum(xr * xr, axis=1, keepdims=True)
    t3 = jnp.sum(xr * agg, axis=1, keepdims=True)
    deg = mi * deg0
    diff = mi * (sqr * deg0 + t2 - 2.0 * t3)
    return jnp.tanh(diff / jnp.maximum(deg, 1.0))


def _g2_kernel(s_hbm, f_ref, w_ref, b_ref, m_ref, out_ref,
               buf_ref, xs_ref, xa_ref, sems):
    B, N, D = f_ref.shape
    nb = N // _BLK
    T = B * nb

    def start(t):
        bb = t // nb
        row0 = (t % nb) * _BLK
        slot = jax.lax.rem(t, _DEPTH)
        pltpu.make_async_copy(
            s_hbm.at[bb, pl.ds(row0, _BLK), :],
            buf_ref.at[slot],
            sems.at[slot],
        ).start()

    for t in range(_DEPTH):
        start(t)

    for bb in range(B):
        x = jnp.dot(f_ref[bb], w_ref[...], preferred_element_type=jnp.float32)
        x = jnp.maximum(x + b_ref[...], 0.0)
        xs_ref[bb] = x
        sq = jnp.sum(x * x, axis=1, keepdims=True)
        lane = jax.lax.broadcasted_iota(jnp.int32, (N, D), 1)
        extra = jnp.where(lane == 0, sq, jnp.where(lane == 1, 1.0, 0.0))
        xa_ref[bb] = (jnp.concatenate([x, extra], axis=1)
                      * m_ref[bb]).astype(jnp.bfloat16)

    def body(t, carry):
        bb = t // nb
        row0 = (t % nb) * _BLK
        slot = jax.lax.rem(t, _DEPTH)
        pltpu.make_async_copy(
            s_hbm.at[bb, pl.ds(row0, _BLK), :],
            buf_ref.at[slot],
            sems.at[slot],
        ).wait()
        xr = xs_ref[bb, pl.ds(row0, _BLK), :]
        mi = m_ref[bb, pl.ds(row0, _BLK), :]
        out_ref[bb, pl.ds(row0, _BLK), :] = _block_tau(
            buf_ref[slot], xa_ref[bb], xr, mi)

        @pl.when(t + _DEPTH < T)
        def _():
            start(t + _DEPTH)

        return carry

    jax.lax.fori_loop(0, T, body, 0)


def kernel(features, support, mask, W, b):
    B, N, D = features.shape
    tau = pl.pallas_call(
        _g2_kernel,
        in_specs=[
            pl.BlockSpec(memory_space=pltpu.MemorySpace.HBM),
            pl.BlockSpec(memory_space=pltpu.MemorySpace.VMEM),
            pl.BlockSpec(memory_space=pltpu.MemorySpace.VMEM),
            pl.BlockSpec(memory_space=pltpu.MemorySpace.VMEM),
            pl.BlockSpec(memory_space=pltpu.MemorySpace.VMEM),
        ],
        out_specs=pl.BlockSpec(memory_space=pltpu.MemorySpace.VMEM),
        out_shape=jax.ShapeDtypeStruct((B, N, 1), jnp.float32),
        scratch_shapes=[
            pltpu.VMEM((_DEPTH, _BLK, N), jnp.float32),
            pltpu.VMEM((B, N, D), jnp.float32),
            pltpu.VMEM((B, N, 2 * D), jnp.bfloat16),
            pltpu.SemaphoreType.DMA((_DEPTH,)),
        ],
    )(support, features, W, b.reshape(1, D), mask)
    return tau
